# baseline (speedup 1.0000x reference)
import functools

import jax
import jax.numpy as jnp
from jax import lax
from jax.experimental import pallas as pl
from jax.experimental.pallas import tpu as pltpu
from jax.experimental.pallas import tpu_sc as plsc

B = 16384
D = 32
V = 1000000
NT = 16
QPT = B // NT
NCH = QPT // 128

_mesh = plsc.VectorSubcoreMesh(core_axis_name="c", subcore_axis_name="s")


def _rsqrt(x):
    i = plsc.bitcast(x, jnp.int32)
    y = plsc.bitcast(jnp.int32(0x5F3759DF) - (i >> 1), jnp.float32)
    for _ in range(3):
        y = y * (1.5 - 0.5 * x * y * y)
    return y


@functools.partial(
    pl.kernel,
    mesh=_mesh,
    out_type=jax.ShapeDtypeStruct((2, B), jnp.float32),
    compiler_params=pltpu.CompilerParams(needs_layout_passes=False),
    scratch_types=[
        pltpu.VMEM((NCH, 128), jnp.int32),
        pltpu.VMEM((NCH, 128), jnp.int32),
        pltpu.VMEM((2, NCH, 128), jnp.int32),
        pltpu.VMEM((2, NCH, 128), jnp.int32),
        pltpu.VMEM((2, NCH, 128), jnp.float32),
        pltpu.VMEM((2, NCH, 128), jnp.float32),
        pltpu.VMEM((QPT,), jnp.float32),
        pltpu.VMEM((QPT,), jnp.float32),
        pltpu.VMEM((QPT,), jnp.float32),
        pltpu.VMEM((D,), jnp.float32),
        pltpu.SemaphoreType.DMA,
        pltpu.SemaphoreType.DMA,
    ],
)
def _sc_kernel(src_hbm, anc_hbm, e1f_hbm, e2f_hbm, rel_hbm,
               out_hbm, sidx, aidx, sib, aib, sbank, abank, numv, ssv, aav,
               relv, sem0, sem1):
    sc = lax.axis_index("c")
    tid = lax.axis_index("s")
    rowbase = tid * NCH

    pltpu.sync_copy(src_hbm.at[pl.ds(rowbase, NCH)], sidx)
    pltpu.sync_copy(anc_hbm.at[pl.ds(rowbase, NCH)], aidx)
    pltpu.sync_copy(rel_hbm.at[sc], relv)

    def zero_body(g, carry):
        q = pl.ds(g * 16, 16)
        z = jnp.zeros((16,), jnp.float32)
        numv[q] = z
        ssv[q] = z
        aav[q] = z
        return carry

    lax.fori_loop(0, QPT // 16, zero_body, 0)

    r_lo = relv[pl.ds(0, 16)]
    r_hi = relv[pl.ds(16, 16)]
    sems = (sem0, sem1)

    def run_table(tbl):
        def stage(st, f):
            foff = jnp.zeros((16,), jnp.int32) + f * V
            for ch in range(NCH):
                for g in range(8):
                    gs = pl.ds(g * 16, 16)
                    sib[st, ch, gs] = sidx[ch, gs] + foff
                    aib[st, ch, gs] = aidx[ch, gs] + foff

        def issue(st):
            for ch in range(NCH):
                pltpu.async_copy(
                    tbl.at[sib.at[st, ch]], sbank.at[st, ch], sems[st])
                pltpu.async_copy(
                    tbl.at[aib.at[st, ch]], abank.at[st, ch], sems[st])

        def waitall(st):
            for ch in range(NCH):
                pltpu.make_async_copy(
                    tbl.at[sib.at[st, 0]], sbank.at[st, ch], sems[st]).wait()
                pltpu.make_async_copy(
                    tbl.at[aib.at[st, 0]], abank.at[st, ch], sems[st]).wait()

        def acc(st, f):
            fv = jnp.zeros((16,), jnp.int32) + (f & 15)
            rbf = jnp.where(
                jnp.zeros((16,), jnp.int32) + f < 16,
                r_lo.at[fv].get(mode="promise_in_bounds"),
                r_hi.at[fv].get(mode="promise_in_bounds"),
            )
            for ch in range(NCH):
                for g in range(8):
                    gs = pl.ds(g * 16, 16)
                    q = pl.ds(ch * 128 + g * 16, 16)
                    s = sbank[st, ch, gs]
                    a = abank[st, ch, gs]
                    v = s * rbf
                    numv[q] = numv[q] + v * a
                    ssv[q] = ssv[q] + v * v
                    aav[q] = aav[q] + a * a

        stage(0, 0)
        issue(0)

        def step(k, carry):
            f = 2 * k
            stage(1, f + 1)
            issue(1)
            waitall(0)
            acc(0, f)

            @pl.when(f + 2 < D)
            def _():
                stage(0, f + 2)
                issue(0)

            waitall(1)
            acc(1, f + 1)
            return carry

        lax.fori_loop(0, D // 2, step, 0)

    @pl.when(sc == 0)
    def _():
        run_table(e1f_hbm)

    @pl.when(sc == 1)
    def _():
        run_table(e2f_hbm)

    def fin_body(g, carry):
        q = pl.ds(g * 16, 16)
        num = numv[q]
        den2 = jnp.maximum(ssv[q] * aav[q], 1e-16)
        numv[q] = num * _rsqrt(den2)
        return carry

    lax.fori_loop(0, QPT // 16, fin_body, 0)
    pltpu.sync_copy(numv, out_hbm.at[sc, pl.ds(tid * QPT, QPT)])


def _combine_body(d_ref, o_ref):
    d1 = d_ref[0]
    d2 = d_ref[1]
    o_ref[...] = 1.0 - (1.0 - d1) * (1.0 - d2)


_combine = pl.pallas_call(
    _combine_body,
    out_shape=jax.ShapeDtypeStruct((128, 128), jnp.float32),
)


def kernel(source_nodes, anchor_nodes, emb1, emb2, rel1, rel2):
    src = source_nodes.astype(jnp.int32).reshape(NT * NCH, 128)
    anc = anchor_nodes.astype(jnp.int32).reshape(NT * NCH, 128)
    relb = jnp.stack([rel1, rel2])
    e1f = emb1.T.reshape(-1)
    e2f = emb2.T.reshape(-1)
    dims = _sc_kernel(src, anc, e1f, e2f, relb)
    return _combine(dims.reshape(2, 128, 128)).reshape(B)


# speedup vs baseline: 28.9791x; 28.9791x over previous
import functools

import jax
import jax.numpy as jnp
from jax import lax
from jax.experimental import pallas as pl
from jax.experimental.pallas import tpu as pltpu
from jax.experimental.pallas import tpu_sc as plsc

B = 16384
D = 32
V = 1000000
NT = 16
QPT = B // NT
NCH = QPT // 128
NGC = 128 // 16

_mesh = plsc.VectorSubcoreMesh(core_axis_name="c", subcore_axis_name="s")


def _rsqrt(x):
    i = plsc.bitcast(x, jnp.int32)
    y = plsc.bitcast(jnp.int32(0x5F3759DF) - (i >> 1), jnp.float32)
    for _ in range(3):
        y = y * (1.5 - 0.5 * x * y * y)
    return y


@functools.partial(
    pl.kernel,
    mesh=_mesh,
    out_type=jax.ShapeDtypeStruct((2, B), jnp.float32),
    compiler_params=pltpu.CompilerParams(needs_layout_passes=False),
    scratch_types=[
        pltpu.VMEM((NCH, 128), jnp.int32),
        pltpu.VMEM((NCH, 128), jnp.int32),
        pltpu.VMEM_SHARED((V,), jnp.float32),
        pltpu.VMEM_SHARED((V,), jnp.float32),
        pltpu.VMEM((2, 128), jnp.float32),
        pltpu.VMEM((2, 128), jnp.float32),
        pltpu.VMEM((QPT,), jnp.float32),
        pltpu.VMEM((QPT,), jnp.float32),
        pltpu.VMEM((QPT,), jnp.float32),
        pltpu.VMEM((D,), jnp.float32),
        pltpu.SemaphoreType.DMA,
        pltpu.SemaphoreType.DMA,
        pltpu.SemaphoreType.DMA,
        pltpu.SemaphoreType.DMA,
        pltpu.SemaphoreType.DMA,
        pltpu.SemaphoreType.DMA,
    ],
)
def _sc_kernel(src_hbm, anc_hbm, e1t_hbm, e2t_hbm, rel_hbm,
               out_hbm, sidx, aidx, rowa, rowb, sbank, abank, numv, ssv, aav,
               relv, sema, semb, gsem0, gsem1, gsem2, gsem3):
    sc = lax.axis_index("c")
    tid = lax.axis_index("s")
    rowbase = tid * NCH

    pltpu.sync_copy(src_hbm.at[pl.ds(rowbase, NCH)], sidx)
    pltpu.sync_copy(anc_hbm.at[pl.ds(rowbase, NCH)], aidx)
    pltpu.sync_copy(rel_hbm.at[sc], relv)

    def zero_body(g, carry):
        q = pl.ds(g * 16, 16)
        z = jnp.zeros((16,), jnp.float32)
        numv[q] = z
        ssv[q] = z
        aav[q] = z
        return carry

    lax.fori_loop(0, QPT // 16, zero_body, 0)

    r_lo = relv[pl.ds(0, 16)]
    r_hi = relv[pl.ds(16, 16)]
    gsems = (gsem0, gsem1, gsem2, gsem3)

    def run_table(tbl):
        @pl.when(tid == 0)
        def _():
            pltpu.async_copy(tbl.at[0], rowa, sema)
            pltpu.async_copy(tbl.at[1], rowb, semb)

        def phase(f, buf, sem):
            @pl.when(tid == 0)
            def _():
                pltpu.make_async_copy(tbl.at[0], buf, sem).wait()

            plsc.subcore_barrier()

            fv = jnp.zeros((16,), jnp.int32) + (f & 15)
            rb = jnp.where(
                jnp.zeros((16,), jnp.int32) + f < 16,
                r_lo.at[fv].get(mode="promise_in_bounds"),
                r_hi.at[fv].get(mode="promise_in_bounds"),
            )

            def issue(ch, b):
                pltpu.async_copy(
                    buf.at[sidx.at[ch]], sbank.at[b], gsems[b])
                pltpu.async_copy(
                    buf.at[aidx.at[ch]], abank.at[b], gsems[b])

            def wait(b):
                pltpu.make_async_copy(
                    buf.at[sidx.at[0]], sbank.at[b], gsems[b]).wait()
                pltpu.make_async_copy(
                    buf.at[aidx.at[0]], abank.at[b], gsems[b]).wait()

            def acc_chunk(ch, b):
                for g in range(NGC):
                    q = pl.ds(ch * 128 + g * 16, 16)
                    s = sbank[b, pl.ds(g * 16, 16)]
                    a = abank[b, pl.ds(g * 16, 16)]
                    v = s * rb
                    numv[q] = numv[q] + v * a
                    ssv[q] = ssv[q] + v * v
                    aav[q] = aav[q] + a * a

            for ch in range(2):
                issue(ch, ch)
            for ch in range(2, NCH):
                b = ch % 2
                wait(b)
                acc_chunk(ch - 2, b)
                issue(ch, b)
            for ch in range(NCH - 2, NCH):
                b = ch % 2
                wait(b)

            plsc.subcore_barrier()

            @pl.when((tid == 0) & (f + 2 < D))
            def _():
                pltpu.async_copy(tbl.at[f + 2], buf, sem)

            for ch in range(NCH - 2, NCH):
                acc_chunk(ch, ch % 2)

        def step(k, carry):
            phase(2 * k, rowa, sema)
            phase(2 * k + 1, rowb, semb)
            return carry

        lax.fori_loop(0, D // 2, step, 0)

    @pl.when(sc == 0)
    def _():
        run_table(e1t_hbm)

    @pl.when(sc == 1)
    def _():
        run_table(e2t_hbm)

    def fin_body(g, carry):
        q = pl.ds(g * 16, 16)
        num = numv[q]
        den2 = jnp.maximum(ssv[q] * aav[q], 1e-16)
        numv[q] = num * _rsqrt(den2)
        return carry

    lax.fori_loop(0, QPT // 16, fin_body, 0)
    pltpu.sync_copy(numv, out_hbm.at[sc, pl.ds(tid * QPT, QPT)])


def _combine_body(d_ref, o_ref):
    d1 = d_ref[0]
    d2 = d_ref[1]
    o_ref[...] = 1.0 - (1.0 - d1) * (1.0 - d2)


_combine = pl.pallas_call(
    _combine_body,
    out_shape=jax.ShapeDtypeStruct((128, 128), jnp.float32),
)


def kernel(source_nodes, anchor_nodes, emb1, emb2, rel1, rel2):
    src = source_nodes.astype(jnp.int32).reshape(NT * NCH, 128)
    anc = anchor_nodes.astype(jnp.int32).reshape(NT * NCH, 128)
    relb = jnp.stack([rel1, rel2])
    dims = _sc_kernel(src, anc, emb1.T, emb2.T, relb)
    return _combine(dims.reshape(2, 128, 128)).reshape(B)
